# Initial kernel scaffold; baseline (speedup 1.0000x reference)
import jax
import jax.numpy as jnp
from jax.experimental import pallas as pl


def kernel(vol, df):
    raise NotImplementedError("write your pallas kernel here")


# speedup vs baseline: 1.1443x; 1.1443x over previous
import functools
import jax
import jax.numpy as jnp
from jax import lax
from jax.experimental import pallas as pl
from jax.experimental.pallas import tpu as pltpu
from jax.experimental.pallas import tpu_sc as plsc

D = H = W = 128
HW = H * W
NVOX = D * HW
NB = 2
NW = 32
CH = 128
VPW = NVOX // NW
NCHUNK = VPW // CH
G = CH // 16

_mesh = plsc.VectorSubcoreMesh(core_axis_name="c", subcore_axis_name="s")


@functools.partial(
    pl.kernel,
    mesh=_mesh,
    out_type=[jax.ShapeDtypeStruct((NVOX,), jnp.float32)] * (NB * 2),
    scratch_types=[
        pltpu.VMEM((CH,), jnp.float32),
        pltpu.VMEM((CH,), jnp.float32),
        pltpu.VMEM((CH,), jnp.float32),
        pltpu.VMEM((8, CH), jnp.int32),
        pltpu.VMEM((8, CH), jnp.float32),
        pltpu.VMEM((8, CH), jnp.float32),
        pltpu.VMEM((8, CH), jnp.float32),
        pltpu.VMEM((CH,), jnp.float32),
        pltpu.VMEM((CH,), jnp.float32),
        pltpu.SemaphoreType.DMA,
        pltpu.SemaphoreType.DMA,
    ],
)
def _sc_warp(v00, v01, v10, v11, d0z, d0y, d0x, d1z, d1y, d1x,
             o00, o01, o10, o11,
             dfz_v, dfy_v, dfx_v, idx_v, w_v, g0_v, g1_v, ov0, ov1,
             sem0, sem1):
    wid = lax.axis_index("s") * 2 + lax.axis_index("c")
    lanes = lax.iota(jnp.int32, 16)
    vols = ((v00, v01), (v10, v11))
    dfs = ((d0z, d0y, d0x), (d1z, d1y, d1x))
    outs = ((o00, o01), (o10, o11))

    for b in range(NB):
        vb0, vb1 = vols[b]
        dbz, dby, dbx = dfs[b]
        ob0, ob1 = outs[b]

        def body(i, _):
            p0 = wid * VPW + i * CH
            pltpu.sync_copy(dbz.at[pl.ds(p0, CH)], dfz_v)
            pltpu.sync_copy(dby.at[pl.ds(p0, CH)], dfy_v)
            pltpu.sync_copy(dbx.at[pl.ds(p0, CH)], dfx_v)

            for g in range(G):
                sl = pl.ds(g * 16, 16)
                p = p0 + g * 16 + lanes
                z = lax.shift_right_logical(p, 14)
                y = jnp.bitwise_and(lax.shift_right_logical(p, 7), 127)
                x = jnp.bitwise_and(p, 127)

                locz = jnp.minimum(jnp.maximum(
                    z.astype(jnp.float32) + dfz_v[sl], 0.0), 127.0)
                locy = jnp.minimum(jnp.maximum(
                    y.astype(jnp.float32) + dfy_v[sl], 0.0), 127.0)
                locx = jnp.minimum(jnp.maximum(
                    x.astype(jnp.float32) + dfx_v[sl], 0.0), 127.0)

                az = jnp.minimum(locz.astype(jnp.int32), 126)
                ay = jnp.minimum(locy.astype(jnp.int32), 126)
                ax = jnp.minimum(locx.astype(jnp.int32), 126)
                tz = locz - az.astype(jnp.float32)
                ty = locy - ay.astype(jnp.float32)
                tx = locx - ax.astype(jnp.float32)
                uz = 1.0 - tz
                uy = 1.0 - ty
                ux = 1.0 - tx

                base = (lax.shift_left(az, 14) + lax.shift_left(ay, 7) + ax)
                w00 = uz * uy
                w01 = uz * ty
                w10 = tz * uy
                w11 = tz * ty

                idx_v[0, sl] = base
                idx_v[1, sl] = base + 1
                idx_v[2, sl] = base + 128
                idx_v[3, sl] = base + 129
                idx_v[4, sl] = base + 16384
                idx_v[5, sl] = base + 16385
                idx_v[6, sl] = base + 16512
                idx_v[7, sl] = base + 16513
                w_v[0, sl] = w00 * ux
                w_v[1, sl] = w00 * tx
                w_v[2, sl] = w01 * ux
                w_v[3, sl] = w01 * tx
                w_v[4, sl] = w10 * ux
                w_v[5, sl] = w10 * tx
                w_v[6, sl] = w11 * ux
                w_v[7, sl] = w11 * tx

            handles = []
            for k in range(8):
                handles.append(
                    pltpu.async_copy(vb0.at[idx_v.at[k]], g0_v.at[k], sem0))
                handles.append(
                    pltpu.async_copy(vb1.at[idx_v.at[k]], g1_v.at[k], sem1))
            for h in handles:
                h.wait()

            for g in range(G):
                sl = pl.ds(g * 16, 16)
                acc0 = w_v[0, sl] * g0_v[0, sl]
                acc1 = w_v[0, sl] * g1_v[0, sl]
                for k in range(1, 8):
                    wk = w_v[k, sl]
                    acc0 = acc0 + wk * g0_v[k, sl]
                    acc1 = acc1 + wk * g1_v[k, sl]
                ov0[sl] = acc0
                ov1[sl] = acc1

            pltpu.sync_copy(ov0, ob0.at[pl.ds(p0, CH)])
            pltpu.sync_copy(ov1, ob1.at[pl.ds(p0, CH)])
            return 0

        lax.fori_loop(0, NCHUNK, body, 0)


def kernel(vol, df):
    v = vol.reshape(NB, 2, NVOX)
    d = df.reshape(NB, 3, NVOX)
    outs = _sc_warp(v[0, 0], v[0, 1], v[1, 0], v[1, 1],
                    d[0, 0], d[0, 1], d[0, 2], d[1, 0], d[1, 1], d[1, 2])
    o00, o01, o10, o11 = outs
    return jnp.stack([jnp.stack([o00, o01]), jnp.stack([o10, o11])]
                     ).reshape(NB, 2, D, H, W)


# speedup vs baseline: 1.5273x; 1.3347x over previous
import functools
import jax
import jax.numpy as jnp
from jax import lax
from jax.experimental import pallas as pl
from jax.experimental.pallas import tpu as pltpu
from jax.experimental.pallas import tpu_sc as plsc

D = H = W = 128
HW = H * W
NVOX = D * HW
NB = 2
NW = 32
CH = 128
VPW = NVOX // NW
NCHUNK = VPW // CH
G = CH // 16

_mesh = plsc.VectorSubcoreMesh(core_axis_name="c", subcore_axis_name="s")


@functools.partial(
    pl.kernel,
    mesh=_mesh,
    out_type=[jax.ShapeDtypeStruct((NVOX,), jnp.float32)] * (NB * 2),
    scratch_types=(
        [pltpu.VMEM((3 * CH,), jnp.float32)] * 2
        + [pltpu.VMEM((8, CH), jnp.int32)] * 2
        + [pltpu.VMEM((8, CH), jnp.float32)] * 2
        + [pltpu.VMEM((8 * CH,), jnp.float32)] * 2
        + [pltpu.VMEM((8 * CH,), jnp.float32)] * 2
        + [pltpu.VMEM((2 * CH,), jnp.float32)] * 2
        + [pltpu.SemaphoreType.DMA] * 8
    ),
)
def _sc_warp(v00, v01, v10, v11, d0z, d0y, d0x, d1z, d1y, d1x,
             o00, o01, o10, o11,
             dfa0, dfa1, idx0, idx1, wv0, wv1, ga0, ga1, gb0, gb1,
             ov0, ov1,
             dfsem0, dfsem1, gasem0, gasem1, gbsem0, gbsem1, osem0, osem1):
    wid = lax.axis_index("s") * 2 + lax.axis_index("c")
    lanes = lax.iota(jnp.int32, 16)
    dfa = (dfa0, dfa1)
    idx = (idx0, idx1)
    wv = (wv0, wv1)
    ga = (ga0, ga1)
    gb = (gb0, gb1)
    ov = (ov0, ov1)
    dfsem = (dfsem0, dfsem1)
    gasem = (gasem0, gasem1)
    gbsem = (gbsem0, gbsem1)
    osem = (osem0, osem1)

    def run_batch(vb0, vb1, dbz, dby, dbx, ob0, ob1, prime):
        def prefetch(s, c):
            p0 = wid * VPW + c * CH
            pltpu.async_copy(dbz.at[pl.ds(p0, CH)],
                             dfa[s].at[pl.ds(0, CH)], dfsem[s])
            pltpu.async_copy(dby.at[pl.ds(p0, CH)],
                             dfa[s].at[pl.ds(CH, CH)], dfsem[s])
            pltpu.async_copy(dbx.at[pl.ds(p0, CH)],
                             dfa[s].at[pl.ds(2 * CH, CH)], dfsem[s])

        def df_drain(s):
            pltpu.make_async_copy(dbz.at[pl.ds(0, 3 * CH)], dfa[s],
                                  dfsem[s]).wait()

        def compute(s, c):
            p0 = wid * VPW + c * CH
            for g in range(G):
                sl = pl.ds(g * 16, 16)
                p = p0 + g * 16 + lanes
                z = lax.shift_right_logical(p, 14)
                y = jnp.bitwise_and(lax.shift_right_logical(p, 7), 127)
                x = jnp.bitwise_and(p, 127)

                locz = jnp.minimum(jnp.maximum(
                    z.astype(jnp.float32) + dfa[s][pl.ds(g * 16, 16)],
                    0.0), 127.0)
                locy = jnp.minimum(jnp.maximum(
                    y.astype(jnp.float32) + dfa[s][pl.ds(CH + g * 16, 16)],
                    0.0), 127.0)
                locx = jnp.minimum(jnp.maximum(
                    x.astype(jnp.float32) + dfa[s][pl.ds(2 * CH + g * 16, 16)],
                    0.0), 127.0)

                az = jnp.minimum(locz.astype(jnp.int32), 126)
                ay = jnp.minimum(locy.astype(jnp.int32), 126)
                ax = jnp.minimum(locx.astype(jnp.int32), 126)
                tz = locz - az.astype(jnp.float32)
                ty = locy - ay.astype(jnp.float32)
                tx = locx - ax.astype(jnp.float32)
                uz = 1.0 - tz
                uy = 1.0 - ty
                ux = 1.0 - tx

                base = (lax.shift_left(az, 14) + lax.shift_left(ay, 7) + ax)
                w00 = uz * uy
                w01 = uz * ty
                w10 = tz * uy
                w11 = tz * ty

                idx[s][0, sl] = base
                idx[s][1, sl] = base + 1
                idx[s][2, sl] = base + 128
                idx[s][3, sl] = base + 129
                idx[s][4, sl] = base + 16384
                idx[s][5, sl] = base + 16385
                idx[s][6, sl] = base + 16512
                idx[s][7, sl] = base + 16513
                wv[s][0, sl] = w00 * ux
                wv[s][1, sl] = w00 * tx
                wv[s][2, sl] = w01 * ux
                wv[s][3, sl] = w01 * tx
                wv[s][4, sl] = w10 * ux
                wv[s][5, sl] = w10 * tx
                wv[s][6, sl] = w11 * ux
                wv[s][7, sl] = w11 * tx

        def gather_enq(s):
            for k in range(8):
                pltpu.async_copy(vb0.at[idx[s].at[k]],
                                 ga[s].at[pl.ds(k * CH, CH)], gasem[s])
                pltpu.async_copy(vb1.at[idx[s].at[k]],
                                 gb[s].at[pl.ds(k * CH, CH)], gbsem[s])

        def stage_a(s, c, pf_c=None):
            df_drain(s)
            compute(s, c)
            gather_enq(s)
            if pf_c is not None:
                prefetch(s, pf_c)

        def out_drain(s):
            pltpu.make_async_copy(ob0.at[pl.ds(0, 2 * CH)], ov[s],
                                  osem[s]).wait()

        def stage_b(s, c):
            pltpu.make_async_copy(vb0.at[pl.ds(0, 8 * CH)], ga[s],
                                  gasem[s]).wait()
            pltpu.make_async_copy(vb1.at[pl.ds(0, 8 * CH)], gb[s],
                                  gbsem[s]).wait()
            out_drain(s)
            for g in range(G):
                sl = pl.ds(g * 16, 16)
                w0 = wv[s][0, sl]
                acc0 = w0 * ga[s][pl.ds(g * 16, 16)]
                acc1 = w0 * gb[s][pl.ds(g * 16, 16)]
                for k in range(1, 8):
                    wk = wv[s][k, sl]
                    acc0 = acc0 + wk * ga[s][pl.ds(k * CH + g * 16, 16)]
                    acc1 = acc1 + wk * gb[s][pl.ds(k * CH + g * 16, 16)]
                ov[s][pl.ds(g * 16, 16)] = acc0
                ov[s][pl.ds(CH + g * 16, 16)] = acc1
            p0 = wid * VPW + c * CH
            pltpu.async_copy(ov[s].at[pl.ds(0, CH)],
                             ob0.at[pl.ds(p0, CH)], osem[s])
            pltpu.async_copy(ov[s].at[pl.ds(CH, CH)],
                             ob1.at[pl.ds(p0, CH)], osem[s])

        if prime:
            for s in range(2):
                pltpu.async_copy(ov[s].at[pl.ds(0, CH)],
                                 ob0.at[pl.ds(wid * VPW, CH)], osem[s])
                pltpu.async_copy(ov[s].at[pl.ds(CH, CH)],
                                 ob1.at[pl.ds(wid * VPW, CH)], osem[s])

        prefetch(0, 0)
        prefetch(1, 1)
        stage_a(0, 0, 2)

        def body(i, carry):
            c1 = 2 * i + 1
            stage_a(1, c1, c1 + 2)
            stage_b(0, c1 - 1)
            c2 = 2 * i + 2
            stage_a(0, c2, jnp.minimum(c2 + 2, NCHUNK - 1))
            stage_b(1, c2 - 1)
            return carry

        lax.fori_loop(0, (NCHUNK - 2) // 2, body, 0)
        stage_a(1, NCHUNK - 1)
        stage_b(0, NCHUNK - 2)
        stage_b(1, NCHUNK - 1)
        df_drain(0)

    run_batch(v00, v01, d0z, d0y, d0x, o00, o01, True)
    run_batch(v10, v11, d1z, d1y, d1x, o10, o11, False)
    pltpu.make_async_copy(o10.at[pl.ds(0, 2 * CH)], ov0, osem0).wait()
    pltpu.make_async_copy(o10.at[pl.ds(0, 2 * CH)], ov1, osem1).wait()


def kernel(vol, df):
    v = vol.reshape(NB, 2, NVOX)
    d = df.reshape(NB, 3, NVOX)
    o00, o01, o10, o11 = _sc_warp(
        v[0, 0], v[0, 1], v[1, 0], v[1, 1],
        d[0, 0], d[0, 1], d[0, 2], d[1, 0], d[1, 1], d[1, 2])
    return jnp.stack([jnp.stack([o00, o01]), jnp.stack([o10, o11])]
                     ).reshape(NB, 2, D, H, W)
